# Initial kernel scaffold; baseline (speedup 1.0000x reference)
import jax
import jax.numpy as jnp
from jax.experimental import pallas as pl


def kernel(x, slab):
    raise NotImplementedError("write your pallas kernel here")


# speedup vs baseline: 1.1797x; 1.1797x over previous
import jax
import jax.numpy as jnp
from jax.experimental import pallas as pl
from jax.experimental.pallas import tpu as pltpu

_IN = 11
_HID = 32
_OUT = 3
_HP = 128
_R_W1, _R_B1, _R_W2, _R_B2 = 0, 16, 24, 152


def _mlp_kernel(x_ref, p_ref, o_ref):
    x = x_ref[...]
    w1 = p_ref[_R_W1:_R_W1 + _IN, :]
    b1 = p_ref[_R_B1:_R_B1 + 1, :]
    w2 = p_ref[_R_W2:_R_W2 + _HP, :]
    b2 = p_ref[_R_B2:_R_B2 + 1, :]

    h = jnp.dot(x, w1, preferred_element_type=jnp.float32) + b1
    h = jnp.maximum(h, 0.0)
    y = jnp.dot(h, w2, preferred_element_type=jnp.float32) + b2
    o_ref[...] = y[:, :_OUT]


def kernel(x, slab):
    B = x.shape[0]
    tb = 4096
    n_steps = pl.cdiv(B, tb)
    pad = n_steps * tb - B
    if pad:
        x = jnp.pad(x, ((0, pad), (0, 0)))

    out = pl.pallas_call(
        _mlp_kernel,
        out_shape=jax.ShapeDtypeStruct((n_steps * tb, _OUT), jnp.float32),
        grid=(n_steps,),
        in_specs=[
            pl.BlockSpec((tb, _IN), lambda i: (i, 0)),
            pl.BlockSpec(slab.shape, lambda i: (0, 0)),
        ],
        out_specs=pl.BlockSpec((tb, _OUT), lambda i: (i, 0)),
        compiler_params=pltpu.CompilerParams(
            dimension_semantics=("parallel",)),
    )(x, slab)
    return out[:B] if pad else out
